# baseline (speedup 1.0000x reference)
import functools

import jax
import jax.numpy as jnp
from jax import lax
from jax.experimental import pallas as pl
from jax.experimental.pallas import tpu as pltpu
from jax.experimental.pallas import tpu_sc as plsc

T = 2048
D = 1024
E = 8
F = 2048
BLK = 256
NT_MOE = 24
PMOE = NT_MOE * BLK
NW = 32
ROWS_W = PMOE // NW
TOK_W = T // NW


def _routing_body(x_ref, wg_ref, bg_ref, wa1_ref, ba1_ref, wa2_ref, ba2_ref,
                  mi_ref, mf_ref, oh1_ref, oh2_ref, r1_ref, r2_ref):
    f32 = jnp.float32
    xv = x_ref[...]
    logits = jnp.dot(xv, wg_ref[...], preferred_element_type=f32) + bg_ref[...]
    m = jnp.max(logits, axis=1, keepdims=True)
    ex = jnp.exp(logits - m)
    probs = ex / jnp.sum(ex, axis=1, keepdims=True)

    idx8 = lax.broadcasted_iota(jnp.int32, (T, E), 1).astype(f32)
    m1 = jnp.max(probs, axis=1, keepdims=True)
    i1 = jnp.min(jnp.where(probs >= m1, idx8, float(E)), axis=1, keepdims=True)
    oh1 = (idx8 == i1).astype(f32)
    pr2 = jnp.where(idx8 == i1, -1.0, probs)
    m2 = jnp.max(pr2, axis=1, keepdims=True)
    i2 = jnp.min(jnp.where(pr2 >= m2, idx8, float(E)), axis=1, keepdims=True)
    oh2 = (idx8 == i2).astype(f32)
    oh1_ref[...] = oh1
    oh2_ref[...] = oh2

    xavg = jnp.mean(xv, axis=0, keepdims=True)
    ha = jnp.maximum(
        jnp.dot(xavg, wa1_ref[...], preferred_element_type=f32) + ba1_ref[...], 0.0)
    z = jnp.dot(ha, wa2_ref[...], preferred_element_type=f32) + ba2_ref[...]
    r = 0.5 / (1.0 + jnp.exp(-z))
    sw = m1 + m2
    w1r = (m1 / sw) * r
    w2r = (m2 / sw) * r

    ls = (lax.broadcasted_iota(jnp.int32, (BLK, BLK), 0)
          > lax.broadcasted_iota(jnp.int32, (BLK, BLK), 1)).astype(f32)

    def body(cidx, carry):
        c1t, c2t = carry
        sl = pl.ds(cidx * BLK, BLK)
        b1 = oh1_ref[sl, :]
        b2 = oh2_ref[sl, :]
        cum1 = jnp.dot(ls, b1, preferred_element_type=f32) + c1t
        cum2 = jnp.dot(ls, b2, preferred_element_type=f32) + c2t
        r1_ref[sl, :] = jnp.sum(cum1 * b1, axis=1, keepdims=True)
        r2_ref[sl, :] = jnp.sum(cum2 * b2, axis=1, keepdims=True)
        return (c1t + jnp.sum(b1, axis=0, keepdims=True),
                c2t + jnp.sum(b2, axis=0, keepdims=True))

    c1t, c2t = lax.fori_loop(
        0, T // BLK, body,
        (jnp.zeros((1, E), f32), jnp.zeros((1, E), f32)))

    counts = c1t + c2t
    cap = jnp.floor((counts + float(BLK - 1)) / float(BLK)) * float(BLK)
    up8 = (lax.broadcasted_iota(jnp.int32, (E, E), 0)
           < lax.broadcasted_iota(jnp.int32, (E, E), 1)).astype(f32)
    gs = jnp.dot(cap, up8, preferred_element_type=f32)
    rank2 = r2_ref[...] + jnp.sum(oh2 * c1t, axis=1, keepdims=True)
    pos0 = jnp.sum(oh1 * gs, axis=1, keepdims=True) + r1_ref[...]
    pos1 = jnp.sum(oh2 * gs, axis=1, keepdims=True) + rank2
    jrow = lax.broadcasted_iota(jnp.int32, (T, 1), 0).astype(f32) * float(BLK)
    te = jnp.sum((gs <= jrow).astype(f32), axis=1, keepdims=True) - 1.0

    mi_ref[...] = jnp.zeros((T, E), jnp.int32)
    mf_ref[...] = jnp.zeros((T, E), f32)
    mi_ref[:, 0:1] = pos0.astype(jnp.int32)
    mi_ref[:, 1:2] = pos1.astype(jnp.int32)
    mi_ref[:, 2:3] = te.astype(jnp.int32)
    mf_ref[:, 0:1] = w1r
    mf_ref[:, 1:2] = w2r
    mf_ref[:, 2:3] = (1.0 - r) + jnp.zeros((T, 1), f32)


def _routing(xf, wg, bg, wa1, ba1, wa2, ba2):
    f32 = jnp.float32
    return pl.pallas_call(
        _routing_body,
        out_shape=[jax.ShapeDtypeStruct((T, E), jnp.int32),
                   jax.ShapeDtypeStruct((T, E), f32)],
        scratch_shapes=[pltpu.VMEM((T, E), f32), pltpu.VMEM((T, E), f32),
                        pltpu.VMEM((T, 1), f32), pltpu.VMEM((T, 1), f32)],
    )(xf, wg, bg.reshape(1, E), wa1, ba1.reshape(1, 64), wa2, ba2.reshape(1, 1))


def _scgather_body(x_hbm, mi_hbm, mf_hbm, xs_hbm, wv_hbm,
                   mi_v, mf_v, ids_v, w_v, idxc, buf, sem, shared_ids):
    i32 = jnp.int32
    c = lax.axis_index("c")
    s = lax.axis_index("s")
    iota16 = lax.iota(i32, 16)
    zeros16 = jnp.zeros((16,), i32)
    ones16 = zeros16 + 1

    @pl.when(s == 0)
    def _phase1():
        pltpu.sync_copy(mi_hbm, mi_v)
        pltpu.sync_copy(mf_hbm, mf_v)

        def initbody(i, carry):
            ids_v[pl.ds(i * 16, 16)] = zeros16
            w_v[pl.ds(i * 16, 16)] = jnp.zeros((16,), jnp.float32)
            return carry

        lax.fori_loop(0, PMOE // 16, initbody, 0)

        def scat(i, carry):
            t16 = iota16 + i * 16
            p0 = plsc.load_gather(mi_v, [t16 * E])
            p1 = plsc.load_gather(mi_v, [t16 * E + 1])
            w0 = plsc.load_gather(mf_v, [t16 * E])
            w1 = plsc.load_gather(mf_v, [t16 * E + 1])
            plsc.store_scatter(ids_v, [p0], t16)
            plsc.store_scatter(ids_v, [p1], t16)
            plsc.store_scatter(w_v, [p0], w0)
            plsc.store_scatter(w_v, [p1], w1)
            return carry

        lax.fori_loop(0, T // 16, scat, 0)
        pltpu.sync_copy(ids_v, shared_ids)

    @pl.when((s == 0) & (c == 0))
    def _wout():
        pltpu.sync_copy(w_v, wv_hbm)

    plsc.subcore_barrier()

    wid = c * 16 + s
    base = wid * ROWS_W
    pltpu.sync_copy(shared_ids.at[pl.ds(base, ROWS_W)], idxc)
    for j in range(ROWS_W // 64):
        pltpu.async_copy(x_hbm.at[idxc.at[pl.ds(j * 64, 64)]], buf, sem).wait()
        pltpu.sync_copy(buf, xs_hbm.at[pl.ds(base + j * 64, 64)])


def _sc_gather(xf, mi, mf):
    f32 = jnp.float32
    i32 = jnp.int32
    mesh = plsc.VectorSubcoreMesh(core_axis_name="c", subcore_axis_name="s")
    kfn = pl.kernel(
        _scgather_body,
        out_type=(jax.ShapeDtypeStruct((PMOE, D), f32),
                  jax.ShapeDtypeStruct((PMOE,), f32)),
        mesh=mesh,
        scratch_types=[
            pltpu.VMEM((T * E,), i32),
            pltpu.VMEM((T * E,), f32),
            pltpu.VMEM((PMOE,), i32),
            pltpu.VMEM((PMOE,), f32),
            pltpu.VMEM((ROWS_W,), i32),
            pltpu.VMEM((64, D), f32),
            pltpu.SemaphoreType.DMA,
            pltpu.VMEM_SHARED((PMOE,), i32),
        ],
        compiler_params=pltpu.CompilerParams(needs_layout_passes=False),
    )
    return kfn(xf, mi, mf)


def _moe_gemm_body(te_ref, xs_ref, w1_ref, b1_ref, w2_ref, b2_ref, wv_ref,
                   out_ref):
    f32 = jnp.float32
    h = jnp.maximum(
        jnp.dot(xs_ref[...], w1_ref[0], preferred_element_type=f32)
        + b1_ref[0], 0.0)
    y = jnp.dot(h, w2_ref[0], preferred_element_type=f32) + b2_ref[0]
    out_ref[...] = y * wv_ref[...]


def _moe_gemm(te, xs, w1, b1, w2, b2, wv):
    grid_spec = pltpu.PrefetchScalarGridSpec(
        num_scalar_prefetch=1,
        grid=(NT_MOE,),
        in_specs=[
            pl.BlockSpec((BLK, D), lambda i, te: (i, 0)),
            pl.BlockSpec((1, D, F), lambda i, te: (te[i], 0, 0)),
            pl.BlockSpec((1, 1, F), lambda i, te: (te[i], 0, 0)),
            pl.BlockSpec((1, F, D), lambda i, te: (te[i], 0, 0)),
            pl.BlockSpec((1, 1, D), lambda i, te: (te[i], 0, 0)),
            pl.BlockSpec((BLK, 1), lambda i, te: (i, 0)),
        ],
        out_specs=pl.BlockSpec((BLK, D), lambda i, te: (i, 0)),
    )
    return pl.pallas_call(
        _moe_gemm_body,
        grid_spec=grid_spec,
        out_shape=jax.ShapeDtypeStruct((PMOE, D), jnp.float32),
        compiler_params=pltpu.CompilerParams(vmem_limit_bytes=50 * 2**20),
    )(te, xs, w1, b1.reshape(E, 1, F), w2, b2.reshape(E, 1, D), wv)


def _dense_body(x_ref, wd1_ref, bd1_ref, wd2_ref, bd2_ref, dr_ref, g0_ref,
                g1_ref, out_ref):
    f32 = jnp.float32
    h = jnp.maximum(
        jnp.dot(x_ref[...], wd1_ref[...], preferred_element_type=f32)
        + bd1_ref[...], 0.0)
    y = jnp.dot(h, wd2_ref[...], preferred_element_type=f32) + bd2_ref[...]
    out_ref[...] = y * dr_ref[...] + g0_ref[...] + g1_ref[...]


def _dense_ffn(xf, wd1, bd1, wd2, bd2, dr11, g0, g1):
    return pl.pallas_call(
        _dense_body,
        grid=(T // BLK,),
        in_specs=[
            pl.BlockSpec((BLK, D), lambda i: (i, 0)),
            pl.BlockSpec((D, F), lambda i: (0, 0)),
            pl.BlockSpec((1, F), lambda i: (0, 0)),
            pl.BlockSpec((F, D), lambda i: (0, 0)),
            pl.BlockSpec((1, D), lambda i: (0, 0)),
            pl.BlockSpec((1, 1), lambda i: (0, 0)),
            pl.BlockSpec((BLK, D), lambda i: (i, 0)),
            pl.BlockSpec((BLK, D), lambda i: (i, 0)),
        ],
        out_specs=pl.BlockSpec((BLK, D), lambda i: (i, 0)),
        out_shape=jax.ShapeDtypeStruct((T, D), jnp.float32),
        compiler_params=pltpu.CompilerParams(vmem_limit_bytes=50 * 2**20),
    )(xf, wd1, bd1.reshape(1, F), wd2, bd2.reshape(1, D), dr11, g0, g1)


def _pairgather_body(ys_hbm, mi_hbm, g0_hbm, g1_hbm, mi_v, p0b, p1b, buf, sem):
    i32 = jnp.int32
    c = lax.axis_index("c")
    s = lax.axis_index("s")
    wid = c * 16 + s
    base = wid * TOK_W
    pltpu.sync_copy(mi_hbm, mi_v)
    iota16 = lax.iota(i32, 16)
    for k in range(TOK_W // 16):
        t16 = iota16 + (base + k * 16)
        p0b[pl.ds(k * 16, 16)] = plsc.load_gather(mi_v, [t16 * E])
        p1b[pl.ds(k * 16, 16)] = plsc.load_gather(mi_v, [t16 * E + 1])
    pltpu.async_copy(ys_hbm.at[p0b], buf, sem).wait()
    pltpu.sync_copy(buf, g0_hbm.at[pl.ds(base, TOK_W)])
    pltpu.async_copy(ys_hbm.at[p1b], buf, sem).wait()
    pltpu.sync_copy(buf, g1_hbm.at[pl.ds(base, TOK_W)])


def _sc_pairgather(ys, mi_flat):
    f32 = jnp.float32
    i32 = jnp.int32
    mesh = plsc.VectorSubcoreMesh(core_axis_name="c", subcore_axis_name="s")
    kfn = pl.kernel(
        _pairgather_body,
        out_type=(jax.ShapeDtypeStruct((T, D), f32),
                  jax.ShapeDtypeStruct((T, D), f32)),
        mesh=mesh,
        scratch_types=[
            pltpu.VMEM((T * E,), i32),
            pltpu.VMEM((TOK_W,), i32),
            pltpu.VMEM((TOK_W,), i32),
            pltpu.VMEM((TOK_W, D), f32),
            pltpu.SemaphoreType.DMA,
        ],
        compiler_params=pltpu.CompilerParams(needs_layout_passes=False),
    )
    return kfn(ys, mi_flat)


def kernel(x, Wg, bg, W1, b1, W2, b2, Wd1, bd1, Wd2, bd2, Wa1, ba1, Wa2, ba2):
    xf = x.reshape(T, D)
    mi, mf = _routing(xf, Wg, bg, Wa1, ba1, Wa2, ba2)
    te = mi[:NT_MOE, 2]
    dr11 = mf[0:1, 2:3]
    xs, wv = _sc_gather(xf, mi.reshape(T * E), mf.reshape(T * E))
    ys = _moe_gemm(te, xs, W1, b1, W2, b2, wv.reshape(PMOE, 1))
    g0, g1 = _sc_pairgather(ys, mi.reshape(T * E))
    out = _dense_ffn(xf, Wd1, bd1, Wd2, bd2, dr11, g0, g1)
    return out.reshape(1, T, D)


# speedup vs baseline: 1.6372x; 1.6372x over previous
import functools

import jax
import jax.numpy as jnp
from jax import lax
from jax.experimental import pallas as pl
from jax.experimental.pallas import tpu as pltpu
from jax.experimental.pallas import tpu_sc as plsc

T = 2048
D = 1024
E = 8
F = 2048
BLK = 256
NT_MOE = 24
PMOE = NT_MOE * BLK
NW = 32
ROWS_W = PMOE // NW
TOK_W = T // NW


def _routing_body(x_ref, wg_ref, bg_ref, wa1_ref, ba1_ref, wa2_ref, ba2_ref,
                  mi_ref, mf_ref, oh1_ref, oh2_ref, r1_ref, r2_ref):
    f32 = jnp.float32
    xv = x_ref[...]
    logits = jnp.dot(xv, wg_ref[...], preferred_element_type=f32) + bg_ref[...]
    m = jnp.max(logits, axis=1, keepdims=True)
    ex = jnp.exp(logits - m)
    probs = ex / jnp.sum(ex, axis=1, keepdims=True)

    idx8 = lax.broadcasted_iota(jnp.int32, (T, E), 1).astype(f32)
    m1 = jnp.max(probs, axis=1, keepdims=True)
    i1 = jnp.min(jnp.where(probs >= m1, idx8, float(E)), axis=1, keepdims=True)
    oh1 = (idx8 == i1).astype(f32)
    pr2 = jnp.where(idx8 == i1, -1.0, probs)
    m2 = jnp.max(pr2, axis=1, keepdims=True)
    i2 = jnp.min(jnp.where(pr2 >= m2, idx8, float(E)), axis=1, keepdims=True)
    oh2 = (idx8 == i2).astype(f32)
    oh1_ref[...] = oh1
    oh2_ref[...] = oh2

    xavg = jnp.mean(xv, axis=0, keepdims=True)
    ha = jnp.maximum(
        jnp.dot(xavg, wa1_ref[...], preferred_element_type=f32) + ba1_ref[...], 0.0)
    z = jnp.dot(ha, wa2_ref[...], preferred_element_type=f32) + ba2_ref[...]
    r = 0.5 / (1.0 + jnp.exp(-z))
    sw = m1 + m2
    w1r = (m1 / sw) * r
    w2r = (m2 / sw) * r

    ls = (lax.broadcasted_iota(jnp.int32, (BLK, BLK), 0)
          > lax.broadcasted_iota(jnp.int32, (BLK, BLK), 1)).astype(f32)

    def body(cidx, carry):
        c1t, c2t = carry
        sl = pl.ds(cidx * BLK, BLK)
        b1 = oh1_ref[sl, :]
        b2 = oh2_ref[sl, :]
        cum1 = jnp.dot(ls, b1, preferred_element_type=f32) + c1t
        cum2 = jnp.dot(ls, b2, preferred_element_type=f32) + c2t
        r1_ref[sl, :] = jnp.sum(cum1 * b1, axis=1, keepdims=True)
        r2_ref[sl, :] = jnp.sum(cum2 * b2, axis=1, keepdims=True)
        return (c1t + jnp.sum(b1, axis=0, keepdims=True),
                c2t + jnp.sum(b2, axis=0, keepdims=True))

    c1t, c2t = lax.fori_loop(
        0, T // BLK, body,
        (jnp.zeros((1, E), f32), jnp.zeros((1, E), f32)))

    counts = c1t + c2t
    cap = jnp.floor((counts + float(BLK - 1)) / float(BLK)) * float(BLK)
    up8 = (lax.broadcasted_iota(jnp.int32, (E, E), 0)
           < lax.broadcasted_iota(jnp.int32, (E, E), 1)).astype(f32)
    gs = jnp.dot(cap, up8, preferred_element_type=f32)
    rank2 = r2_ref[...] + jnp.sum(oh2 * c1t, axis=1, keepdims=True)
    pos0 = jnp.sum(oh1 * gs, axis=1, keepdims=True) + r1_ref[...]
    pos1 = jnp.sum(oh2 * gs, axis=1, keepdims=True) + rank2
    jrow = lax.broadcasted_iota(jnp.int32, (T, 1), 0).astype(f32) * float(BLK)
    te = jnp.sum((gs <= jrow).astype(f32), axis=1, keepdims=True) - 1.0

    mi_ref[...] = jnp.zeros((T, E), jnp.int32)
    mf_ref[...] = jnp.zeros((T, E), f32)
    mi_ref[:, 0:1] = pos0.astype(jnp.int32)
    mi_ref[:, 1:2] = pos1.astype(jnp.int32)
    mi_ref[:, 2:3] = te.astype(jnp.int32)
    mf_ref[:, 0:1] = w1r
    mf_ref[:, 1:2] = w2r
    mf_ref[:, 2:3] = (1.0 - r) + jnp.zeros((T, 1), f32)


def _routing(xf, wg, bg, wa1, ba1, wa2, ba2):
    f32 = jnp.float32
    return pl.pallas_call(
        _routing_body,
        out_shape=[jax.ShapeDtypeStruct((T, E), jnp.int32),
                   jax.ShapeDtypeStruct((T, E), f32)],
        scratch_shapes=[pltpu.VMEM((T, E), f32), pltpu.VMEM((T, E), f32),
                        pltpu.VMEM((T, 1), f32), pltpu.VMEM((T, 1), f32)],
    )(xf, wg, bg.reshape(1, E), wa1, ba1.reshape(1, 64), wa2, ba2.reshape(1, 1))


def _scdispatch_body(x_hbm, p0_hbm, p1_hbm, xs_hbm, p0b, p1b, xbuf, sem, sem2):
    c = lax.axis_index("c")
    s = lax.axis_index("s")
    wid = c * 16 + s
    base = wid * TOK_W
    pltpu.sync_copy(x_hbm.at[pl.ds(base, TOK_W)], xbuf)
    pltpu.sync_copy(p0_hbm.at[pl.ds(base, TOK_W)], p0b)
    pltpu.sync_copy(p1_hbm.at[pl.ds(base, TOK_W)], p1b)
    a = pltpu.async_copy(xbuf, xs_hbm.at[p0b], sem)
    b = pltpu.async_copy(xbuf, xs_hbm.at[p1b], sem2)
    a.wait()
    b.wait()


def _sc_dispatch(xf, p0, p1):
    f32 = jnp.float32
    i32 = jnp.int32
    mesh = plsc.VectorSubcoreMesh(core_axis_name="c", subcore_axis_name="s")
    kfn = pl.kernel(
        _scdispatch_body,
        out_type=jax.ShapeDtypeStruct((PMOE, D), f32),
        mesh=mesh,
        scratch_types=[
            pltpu.VMEM((TOK_W,), i32),
            pltpu.VMEM((TOK_W,), i32),
            pltpu.VMEM((TOK_W, D), f32),
            pltpu.SemaphoreType.DMA,
            pltpu.SemaphoreType.DMA,
        ],
        compiler_params=pltpu.CompilerParams(needs_layout_passes=False),
    )
    return kfn(xf, p0, p1)


def _moe_gemm_body(te_ref, xs_ref, w1_ref, b1_ref, w2_ref, b2_ref, out_ref):
    f32 = jnp.float32
    h = jnp.maximum(
        jnp.dot(xs_ref[...], w1_ref[0], preferred_element_type=f32)
        + b1_ref[0], 0.0)
    out_ref[...] = jnp.dot(h, w2_ref[0], preferred_element_type=f32) + b2_ref[0]


def _moe_gemm(te, xs, w1, b1, w2, b2):
    grid_spec = pltpu.PrefetchScalarGridSpec(
        num_scalar_prefetch=1,
        grid=(NT_MOE,),
        in_specs=[
            pl.BlockSpec((BLK, D), lambda i, te: (i, 0)),
            pl.BlockSpec((1, D, F), lambda i, te: (te[i], 0, 0)),
            pl.BlockSpec((1, 1, F), lambda i, te: (te[i], 0, 0)),
            pl.BlockSpec((1, F, D), lambda i, te: (te[i], 0, 0)),
            pl.BlockSpec((1, 1, D), lambda i, te: (te[i], 0, 0)),
        ],
        out_specs=pl.BlockSpec((BLK, D), lambda i, te: (i, 0)),
    )
    return pl.pallas_call(
        _moe_gemm_body,
        grid_spec=grid_spec,
        out_shape=jax.ShapeDtypeStruct((PMOE, D), jnp.float32),
        compiler_params=pltpu.CompilerParams(vmem_limit_bytes=50 * 2**20),
    )(te, xs, w1, b1.reshape(E, 1, F), w2, b2.reshape(E, 1, D))


def _dense_body(x_ref, wd1_ref, bd1_ref, wd2_ref, bd2_ref, mf_ref, g0_ref,
                g1_ref, out_ref):
    f32 = jnp.float32
    h = jnp.maximum(
        jnp.dot(x_ref[...], wd1_ref[...], preferred_element_type=f32)
        + bd1_ref[...], 0.0)
    y = jnp.dot(h, wd2_ref[...], preferred_element_type=f32) + bd2_ref[...]
    out_ref[...] = (y * mf_ref[:, 2:3] + g0_ref[...] * mf_ref[:, 0:1]
                    + g1_ref[...] * mf_ref[:, 1:2])


def _dense_ffn(xf, wd1, bd1, wd2, bd2, mf, g0, g1):
    return pl.pallas_call(
        _dense_body,
        grid=(T // BLK,),
        in_specs=[
            pl.BlockSpec((BLK, D), lambda i: (i, 0)),
            pl.BlockSpec((D, F), lambda i: (0, 0)),
            pl.BlockSpec((1, F), lambda i: (0, 0)),
            pl.BlockSpec((F, D), lambda i: (0, 0)),
            pl.BlockSpec((1, D), lambda i: (0, 0)),
            pl.BlockSpec((BLK, E), lambda i: (i, 0)),
            pl.BlockSpec((BLK, D), lambda i: (i, 0)),
            pl.BlockSpec((BLK, D), lambda i: (i, 0)),
        ],
        out_specs=pl.BlockSpec((BLK, D), lambda i: (i, 0)),
        out_shape=jax.ShapeDtypeStruct((T, D), jnp.float32),
        compiler_params=pltpu.CompilerParams(vmem_limit_bytes=50 * 2**20),
    )(xf, wd1, bd1.reshape(1, F), wd2, bd2.reshape(1, D), mf, g0, g1)


def _pairgather_body(ys_hbm, p0_hbm, p1_hbm, g0_hbm, g1_hbm, p0b, p1b, buf,
                     sem, sem2):
    c = lax.axis_index("c")
    s = lax.axis_index("s")
    wid = c * 16 + s
    base = wid * TOK_W
    pltpu.sync_copy(p0_hbm.at[pl.ds(base, TOK_W)], p0b)
    pltpu.sync_copy(p1_hbm.at[pl.ds(base, TOK_W)], p1b)
    pltpu.async_copy(ys_hbm.at[p0b], buf, sem).wait()
    pltpu.sync_copy(buf, g0_hbm.at[pl.ds(base, TOK_W)])
    pltpu.async_copy(ys_hbm.at[p1b], buf, sem2).wait()
    pltpu.sync_copy(buf, g1_hbm.at[pl.ds(base, TOK_W)])


def _sc_pairgather(ys, p0, p1):
    f32 = jnp.float32
    i32 = jnp.int32
    mesh = plsc.VectorSubcoreMesh(core_axis_name="c", subcore_axis_name="s")
    kfn = pl.kernel(
        _pairgather_body,
        out_type=(jax.ShapeDtypeStruct((T, D), f32),
                  jax.ShapeDtypeStruct((T, D), f32)),
        mesh=mesh,
        scratch_types=[
            pltpu.VMEM((TOK_W,), i32),
            pltpu.VMEM((TOK_W,), i32),
            pltpu.VMEM((TOK_W, D), f32),
            pltpu.SemaphoreType.DMA,
            pltpu.SemaphoreType.DMA,
        ],
        compiler_params=pltpu.CompilerParams(needs_layout_passes=False),
    )
    return kfn(ys, p0, p1)


def kernel(x, Wg, bg, W1, b1, W2, b2, Wd1, bd1, Wd2, bd2, Wa1, ba1, Wa2, ba2):
    xf = x.reshape(T, D)
    mi, mf = _routing(xf, Wg, bg, Wa1, ba1, Wa2, ba2)
    te = mi[:NT_MOE, 2]
    p0 = mi[:, 0]
    p1 = mi[:, 1]
    xs = _sc_dispatch(xf, p0, p1)
    ys = _moe_gemm(te, xs, W1, b1, W2, b2)
    g0, g1 = _sc_pairgather(ys, p0, p1)
    out = _dense_ffn(xf, Wd1, bd1, Wd2, bd2, mf, g0, g1)
    return out.reshape(1, T, D)


# speedup vs baseline: 1.7403x; 1.0630x over previous
import functools

import jax
import jax.numpy as jnp
from jax import lax
from jax.experimental import pallas as pl
from jax.experimental.pallas import tpu as pltpu
from jax.experimental.pallas import tpu_sc as plsc

T = 2048
D = 1024
E = 8
F = 2048
BLK = 256
NT_MOE = 24
PMOE = NT_MOE * BLK
NW = 32
ROWS_W = PMOE // NW
TOK_W = T // NW


def _routing_body(x_ref, wg_ref, bg_ref, wa1_ref, ba1_ref, wa2_ref, ba2_ref,
                  mi_ref, mf_ref, oh1_ref, oh2_ref, r1_ref, r2_ref):
    f32 = jnp.float32
    xv = x_ref[...]
    logits = jnp.dot(xv, wg_ref[...], preferred_element_type=f32) + bg_ref[...]
    m = jnp.max(logits, axis=1, keepdims=True)
    ex = jnp.exp(logits - m)
    probs = ex / jnp.sum(ex, axis=1, keepdims=True)

    idx8 = lax.broadcasted_iota(jnp.int32, (T, E), 1).astype(f32)
    m1 = jnp.max(probs, axis=1, keepdims=True)
    i1 = jnp.min(jnp.where(probs >= m1, idx8, float(E)), axis=1, keepdims=True)
    oh1 = (idx8 == i1).astype(f32)
    pr2 = jnp.where(idx8 == i1, -1.0, probs)
    m2 = jnp.max(pr2, axis=1, keepdims=True)
    i2 = jnp.min(jnp.where(pr2 >= m2, idx8, float(E)), axis=1, keepdims=True)
    oh2 = (idx8 == i2).astype(f32)
    oh1_ref[...] = oh1
    oh2_ref[...] = oh2

    xavg = jnp.mean(xv, axis=0, keepdims=True)
    ha = jnp.maximum(
        jnp.dot(xavg, wa1_ref[...], preferred_element_type=f32) + ba1_ref[...], 0.0)
    z = jnp.dot(ha, wa2_ref[...], preferred_element_type=f32) + ba2_ref[...]
    r = 0.5 / (1.0 + jnp.exp(-z))
    sw = m1 + m2
    w1r = (m1 / sw) * r
    w2r = (m2 / sw) * r

    ls = (lax.broadcasted_iota(jnp.int32, (BLK, BLK), 0)
          > lax.broadcasted_iota(jnp.int32, (BLK, BLK), 1)).astype(f32)

    def body(cidx, carry):
        c1t, c2t = carry
        sl = pl.ds(cidx * BLK, BLK)
        b1 = oh1_ref[sl, :]
        b2 = oh2_ref[sl, :]
        cum1 = jnp.dot(ls, b1, preferred_element_type=f32) + c1t
        cum2 = jnp.dot(ls, b2, preferred_element_type=f32) + c2t
        r1_ref[sl, :] = jnp.sum(cum1 * b1, axis=1, keepdims=True)
        r2_ref[sl, :] = jnp.sum(cum2 * b2, axis=1, keepdims=True)
        return (c1t + jnp.sum(b1, axis=0, keepdims=True),
                c2t + jnp.sum(b2, axis=0, keepdims=True))

    c1t, c2t = lax.fori_loop(
        0, T // BLK, body,
        (jnp.zeros((1, E), f32), jnp.zeros((1, E), f32)))

    counts = c1t + c2t
    cap = jnp.floor((counts + float(BLK - 1)) / float(BLK)) * float(BLK)
    up8 = (lax.broadcasted_iota(jnp.int32, (E, E), 0)
           < lax.broadcasted_iota(jnp.int32, (E, E), 1)).astype(f32)
    gs = jnp.dot(cap, up8, preferred_element_type=f32)
    rank2 = r2_ref[...] + jnp.sum(oh2 * c1t, axis=1, keepdims=True)
    pos0 = jnp.sum(oh1 * gs, axis=1, keepdims=True) + r1_ref[...]
    pos1 = jnp.sum(oh2 * gs, axis=1, keepdims=True) + rank2
    jrowi = lax.broadcasted_iota(jnp.int32, (T, 1), 0).astype(f32)
    jrow = jrowi * float(BLK)
    te = jnp.sum((gs <= jrow).astype(f32), axis=1, keepdims=True) - 1.0
    tot = jnp.sum(cap, axis=1, keepdims=True)
    nact = jnp.floor((tot + float(BLK - 1)) / float(BLK))
    va = jnp.minimum(jrowi, nact - 1.0)

    mi_ref[...] = jnp.zeros((T, E), jnp.int32)
    mf_ref[...] = jnp.zeros((T, E), f32)
    mi_ref[:, 0:1] = pos0.astype(jnp.int32)
    mi_ref[:, 1:2] = pos1.astype(jnp.int32)
    mi_ref[:, 2:3] = te.astype(jnp.int32)
    mi_ref[:, 3:4] = va.astype(jnp.int32)
    mf_ref[:, 0:1] = w1r
    mf_ref[:, 1:2] = w2r
    mf_ref[:, 2:3] = (1.0 - r) + jnp.zeros((T, 1), f32)


def _routing(xf, wg, bg, wa1, ba1, wa2, ba2):
    f32 = jnp.float32
    return pl.pallas_call(
        _routing_body,
        out_shape=[jax.ShapeDtypeStruct((T, E), jnp.int32),
                   jax.ShapeDtypeStruct((T, E), f32)],
        scratch_shapes=[pltpu.VMEM((T, E), f32), pltpu.VMEM((T, E), f32),
                        pltpu.VMEM((T, 1), f32), pltpu.VMEM((T, 1), f32)],
    )(xf, wg, bg.reshape(1, E), wa1, ba1.reshape(1, 64), wa2, ba2.reshape(1, 1))


def _scdispatch_body(x_hbm, mi_hbm, xs_hbm, p0b, p1b, mchunk, xbuf, sem, sem2):
    i32 = jnp.int32
    c = lax.axis_index("c")
    s = lax.axis_index("s")
    wid = c * 16 + s
    base = wid * TOK_W
    pltpu.sync_copy(x_hbm.at[pl.ds(base, TOK_W)], xbuf)
    pltpu.sync_copy(mi_hbm.at[pl.ds(base * E, TOK_W * E)], mchunk)
    iota16 = lax.iota(i32, 16)
    for k in range(TOK_W // 16):
        i16 = (iota16 + k * 16) * E
        p0b[pl.ds(k * 16, 16)] = plsc.load_gather(mchunk, [i16])
        p1b[pl.ds(k * 16, 16)] = plsc.load_gather(mchunk, [i16 + 1])
    a = pltpu.async_copy(xbuf, xs_hbm.at[p0b], sem)
    b = pltpu.async_copy(xbuf, xs_hbm.at[p1b], sem2)
    a.wait()
    b.wait()


def _sc_dispatch(xf, mi_flat):
    f32 = jnp.float32
    i32 = jnp.int32
    mesh = plsc.VectorSubcoreMesh(core_axis_name="c", subcore_axis_name="s")
    kfn = pl.kernel(
        _scdispatch_body,
        out_type=jax.ShapeDtypeStruct((PMOE, D), f32),
        mesh=mesh,
        scratch_types=[
            pltpu.VMEM((TOK_W,), i32),
            pltpu.VMEM((TOK_W,), i32),
            pltpu.VMEM((TOK_W * E,), i32),
            pltpu.VMEM((TOK_W, D), f32),
            pltpu.SemaphoreType.DMA,
            pltpu.SemaphoreType.DMA,
        ],
        compiler_params=pltpu.CompilerParams(needs_layout_passes=False),
    )
    return kfn(xf, mi_flat)


def _moe_gemm_body(te_ref, va_ref, xs_ref, w1_ref, b1_ref, w2_ref, b2_ref,
                   out_ref):
    f32 = jnp.float32
    i = pl.program_id(0)

    @pl.when(va_ref[i] == i)
    def _compute():
        h = jnp.maximum(
            jnp.dot(xs_ref[...], w1_ref[0], preferred_element_type=f32)
            + b1_ref[0], 0.0)
        out_ref[...] = (jnp.dot(h, w2_ref[0], preferred_element_type=f32)
                        + b2_ref[0])


def _moe_gemm(te, va, xs, w1, b1, w2, b2):
    grid_spec = pltpu.PrefetchScalarGridSpec(
        num_scalar_prefetch=2,
        grid=(NT_MOE,),
        in_specs=[
            pl.BlockSpec((BLK, D), lambda i, te, va: (va[i], 0)),
            pl.BlockSpec((1, D, F), lambda i, te, va: (te[va[i]], 0, 0)),
            pl.BlockSpec((1, 1, F), lambda i, te, va: (te[va[i]], 0, 0)),
            pl.BlockSpec((1, F, D), lambda i, te, va: (te[va[i]], 0, 0)),
            pl.BlockSpec((1, 1, D), lambda i, te, va: (te[va[i]], 0, 0)),
        ],
        out_specs=pl.BlockSpec((BLK, D), lambda i, te, va: (va[i], 0)),
    )
    return pl.pallas_call(
        _moe_gemm_body,
        grid_spec=grid_spec,
        out_shape=jax.ShapeDtypeStruct((PMOE, D), jnp.float32),
        compiler_params=pltpu.CompilerParams(vmem_limit_bytes=50 * 2**20),
    )(te, va, xs, w1, b1.reshape(E, 1, F), w2, b2.reshape(E, 1, D))


def _dense_body(x_ref, wd1_ref, bd1_ref, wd2_ref, bd2_ref, mf_ref, g0_ref,
                g1_ref, out_ref):
    f32 = jnp.float32
    h = jnp.maximum(
        jnp.dot(x_ref[...], wd1_ref[...], preferred_element_type=f32)
        + bd1_ref[...], 0.0)
    y = jnp.dot(h, wd2_ref[...], preferred_element_type=f32) + bd2_ref[...]
    out_ref[...] = (y * mf_ref[:, 2:3] + g0_ref[...] * mf_ref[:, 0:1]
                    + g1_ref[...] * mf_ref[:, 1:2])


def _dense_ffn(xf, wd1, bd1, wd2, bd2, mf, g0, g1):
    return pl.pallas_call(
        _dense_body,
        grid=(T // BLK,),
        in_specs=[
            pl.BlockSpec((BLK, D), lambda i: (i, 0)),
            pl.BlockSpec((D, F), lambda i: (0, 0)),
            pl.BlockSpec((1, F), lambda i: (0, 0)),
            pl.BlockSpec((F, D), lambda i: (0, 0)),
            pl.BlockSpec((1, D), lambda i: (0, 0)),
            pl.BlockSpec((BLK, E), lambda i: (i, 0)),
            pl.BlockSpec((BLK, D), lambda i: (i, 0)),
            pl.BlockSpec((BLK, D), lambda i: (i, 0)),
        ],
        out_specs=pl.BlockSpec((BLK, D), lambda i: (i, 0)),
        out_shape=jax.ShapeDtypeStruct((T, D), jnp.float32),
        compiler_params=pltpu.CompilerParams(vmem_limit_bytes=50 * 2**20),
    )(xf, wd1, bd1.reshape(1, F), wd2, bd2.reshape(1, D), mf, g0, g1)


def _pairgather_body(ys_hbm, mi_hbm, g0_hbm, g1_hbm, p0b, p1b, mchunk, buf,
                     sem, sem2):
    i32 = jnp.int32
    c = lax.axis_index("c")
    s = lax.axis_index("s")
    wid = c * 16 + s
    base = wid * TOK_W
    pltpu.sync_copy(mi_hbm.at[pl.ds(base * E, TOK_W * E)], mchunk)
    iota16 = lax.iota(i32, 16)
    for k in range(TOK_W // 16):
        i16 = (iota16 + k * 16) * E
        p0b[pl.ds(k * 16, 16)] = plsc.load_gather(mchunk, [i16])
        p1b[pl.ds(k * 16, 16)] = plsc.load_gather(mchunk, [i16 + 1])
    pltpu.async_copy(ys_hbm.at[p0b], buf, sem).wait()
    pltpu.sync_copy(buf, g0_hbm.at[pl.ds(base, TOK_W)])
    pltpu.async_copy(ys_hbm.at[p1b], buf, sem2).wait()
    pltpu.sync_copy(buf, g1_hbm.at[pl.ds(base, TOK_W)])


def _sc_pairgather(ys, mi_flat):
    f32 = jnp.float32
    i32 = jnp.int32
    mesh = plsc.VectorSubcoreMesh(core_axis_name="c", subcore_axis_name="s")
    kfn = pl.kernel(
        _pairgather_body,
        out_type=(jax.ShapeDtypeStruct((T, D), f32),
                  jax.ShapeDtypeStruct((T, D), f32)),
        mesh=mesh,
        scratch_types=[
            pltpu.VMEM((TOK_W,), i32),
            pltpu.VMEM((TOK_W,), i32),
            pltpu.VMEM((TOK_W * E,), i32),
            pltpu.VMEM((TOK_W, D), f32),
            pltpu.SemaphoreType.DMA,
            pltpu.SemaphoreType.DMA,
        ],
        compiler_params=pltpu.CompilerParams(needs_layout_passes=False),
    )
    return kfn(ys, mi_flat)


def kernel(x, Wg, bg, W1, b1, W2, b2, Wd1, bd1, Wd2, bd2, Wa1, ba1, Wa2, ba2):
    xf = x.reshape(T, D)
    mi, mf = _routing(xf, Wg, bg, Wa1, ba1, Wa2, ba2)
    te = mi[:NT_MOE, 2]
    va = mi[:NT_MOE, 3]
    mif = mi.reshape(T * E)
    xs = _sc_dispatch(xf, mif)
    ys = _moe_gemm(te, va, xs, W1, b1, W2, b2)
    g0, g1 = _sc_pairgather(ys, mif)
    out = _dense_ffn(xf, Wd1, bd1, Wd2, bd2, mf, g0, g1)
    return out.reshape(1, T, D)


# speedup vs baseline: 1.8580x; 1.0676x over previous
import functools

import jax
import jax.numpy as jnp
from jax import lax
from jax.experimental import pallas as pl
from jax.experimental.pallas import tpu as pltpu
from jax.experimental.pallas import tpu_sc as plsc

T = 2048
D = 1024
E = 8
F = 2048
BLK = 512
NT_MOE = 16
PMOE = NT_MOE * BLK
NW = 32
ROWS_W = PMOE // NW
TOK_W = T // NW


def _routing_body(x_ref, wg_ref, bg_ref, wa1_ref, ba1_ref, wa2_ref, ba2_ref,
                  mi_ref, mf_ref, oh1_ref, oh2_ref, r1_ref, r2_ref):
    f32 = jnp.float32
    xv = x_ref[...]
    logits = jnp.dot(xv, wg_ref[...], preferred_element_type=f32) + bg_ref[...]
    m = jnp.max(logits, axis=1, keepdims=True)
    ex = jnp.exp(logits - m)
    probs = ex / jnp.sum(ex, axis=1, keepdims=True)

    idx8 = lax.broadcasted_iota(jnp.int32, (T, E), 1).astype(f32)
    m1 = jnp.max(probs, axis=1, keepdims=True)
    i1 = jnp.min(jnp.where(probs >= m1, idx8, float(E)), axis=1, keepdims=True)
    oh1 = (idx8 == i1).astype(f32)
    pr2 = jnp.where(idx8 == i1, -1.0, probs)
    m2 = jnp.max(pr2, axis=1, keepdims=True)
    i2 = jnp.min(jnp.where(pr2 >= m2, idx8, float(E)), axis=1, keepdims=True)
    oh2 = (idx8 == i2).astype(f32)
    oh1_ref[...] = oh1
    oh2_ref[...] = oh2

    xavg = jnp.mean(xv, axis=0, keepdims=True)
    ha = jnp.maximum(
        jnp.dot(xavg, wa1_ref[...], preferred_element_type=f32) + ba1_ref[...], 0.0)
    z = jnp.dot(ha, wa2_ref[...], preferred_element_type=f32) + ba2_ref[...]
    r = 0.5 / (1.0 + jnp.exp(-z))
    sw = m1 + m2
    w1r = (m1 / sw) * r
    w2r = (m2 / sw) * r

    ls = (lax.broadcasted_iota(jnp.int32, (BLK, BLK), 0)
          > lax.broadcasted_iota(jnp.int32, (BLK, BLK), 1)).astype(f32)

    def body(cidx, carry):
        c1t, c2t = carry
        sl = pl.ds(cidx * BLK, BLK)
        b1 = oh1_ref[sl, :]
        b2 = oh2_ref[sl, :]
        cum1 = jnp.dot(ls, b1, preferred_element_type=f32) + c1t
        cum2 = jnp.dot(ls, b2, preferred_element_type=f32) + c2t
        r1_ref[sl, :] = jnp.sum(cum1 * b1, axis=1, keepdims=True)
        r2_ref[sl, :] = jnp.sum(cum2 * b2, axis=1, keepdims=True)
        return (c1t + jnp.sum(b1, axis=0, keepdims=True),
                c2t + jnp.sum(b2, axis=0, keepdims=True))

    c1t, c2t = lax.fori_loop(
        0, T // BLK, body,
        (jnp.zeros((1, E), f32), jnp.zeros((1, E), f32)))

    counts = c1t + c2t
    cap = jnp.floor((counts + float(BLK - 1)) / float(BLK)) * float(BLK)
    up8 = (lax.broadcasted_iota(jnp.int32, (E, E), 0)
           < lax.broadcasted_iota(jnp.int32, (E, E), 1)).astype(f32)
    gs = jnp.dot(cap, up8, preferred_element_type=f32)
    rank2 = r2_ref[...] + jnp.sum(oh2 * c1t, axis=1, keepdims=True)
    pos0 = jnp.sum(oh1 * gs, axis=1, keepdims=True) + r1_ref[...]
    pos1 = jnp.sum(oh2 * gs, axis=1, keepdims=True) + rank2
    jrowi = lax.broadcasted_iota(jnp.int32, (T, 1), 0).astype(f32)
    jrow = jrowi * float(BLK)
    te = jnp.sum((gs <= jrow).astype(f32), axis=1, keepdims=True) - 1.0
    tot = jnp.sum(cap, axis=1, keepdims=True)
    nact = jnp.floor((tot + float(BLK - 1)) / float(BLK))
    va = jnp.minimum(jrowi, nact - 1.0)

    mi_ref[...] = jnp.zeros((T, E), jnp.int32)
    mf_ref[...] = jnp.zeros((T, E), f32)
    mi_ref[:, 0:1] = pos0.astype(jnp.int32)
    mi_ref[:, 1:2] = pos1.astype(jnp.int32)
    mi_ref[:, 2:3] = te.astype(jnp.int32)
    mi_ref[:, 3:4] = va.astype(jnp.int32)
    mf_ref[:, 0:1] = w1r
    mf_ref[:, 1:2] = w2r
    mf_ref[:, 2:3] = (1.0 - r) + jnp.zeros((T, 1), f32)


def _routing(xf, wg, bg, wa1, ba1, wa2, ba2):
    f32 = jnp.float32
    return pl.pallas_call(
        _routing_body,
        out_shape=[jax.ShapeDtypeStruct((T, E), jnp.int32),
                   jax.ShapeDtypeStruct((T, E), f32)],
        scratch_shapes=[pltpu.VMEM((T, E), f32), pltpu.VMEM((T, E), f32),
                        pltpu.VMEM((T, 1), f32), pltpu.VMEM((T, 1), f32)],
    )(xf, wg, bg.reshape(1, E), wa1, ba1.reshape(1, 64), wa2, ba2.reshape(1, 1))


def _scdispatch_body(x_hbm, mi_hbm, xs_hbm, p0b, p1b, mchunk, xbuf, sem, sem2):
    i32 = jnp.int32
    c = lax.axis_index("c")
    s = lax.axis_index("s")
    wid = c * 16 + s
    base = wid * TOK_W
    pltpu.sync_copy(x_hbm.at[pl.ds(base, TOK_W)], xbuf)
    pltpu.sync_copy(mi_hbm.at[pl.ds(base * E, TOK_W * E)], mchunk)
    iota16 = lax.iota(i32, 16)
    for k in range(TOK_W // 16):
        i16 = (iota16 + k * 16) * E
        p0b[pl.ds(k * 16, 16)] = plsc.load_gather(mchunk, [i16])
        p1b[pl.ds(k * 16, 16)] = plsc.load_gather(mchunk, [i16 + 1])
    a = pltpu.async_copy(xbuf, xs_hbm.at[p0b], sem)
    b = pltpu.async_copy(xbuf, xs_hbm.at[p1b], sem2)
    a.wait()
    b.wait()


def _sc_dispatch(xf, mi_flat):
    f32 = jnp.float32
    i32 = jnp.int32
    mesh = plsc.VectorSubcoreMesh(core_axis_name="c", subcore_axis_name="s")
    kfn = pl.kernel(
        _scdispatch_body,
        out_type=jax.ShapeDtypeStruct((PMOE, D), f32),
        mesh=mesh,
        scratch_types=[
            pltpu.VMEM((TOK_W,), i32),
            pltpu.VMEM((TOK_W,), i32),
            pltpu.VMEM((TOK_W * E,), i32),
            pltpu.VMEM((TOK_W, D), f32),
            pltpu.SemaphoreType.DMA,
            pltpu.SemaphoreType.DMA,
        ],
        compiler_params=pltpu.CompilerParams(needs_layout_passes=False),
    )
    return kfn(xf, mi_flat)


def _moe_gemm_body(te_ref, va_ref, xs_ref, w1_ref, b1_ref, w2_ref, b2_ref,
                   out_ref):
    f32 = jnp.float32
    i = pl.program_id(0)

    @pl.when(va_ref[i] == i)
    def _compute():
        h = jnp.maximum(
            jnp.dot(xs_ref[...], w1_ref[0], preferred_element_type=f32)
            + b1_ref[0], 0.0)
        out_ref[...] = (jnp.dot(h, w2_ref[0], preferred_element_type=f32)
                        + b2_ref[0])


def _moe_gemm(te, va, xs, w1, b1, w2, b2):
    grid_spec = pltpu.PrefetchScalarGridSpec(
        num_scalar_prefetch=2,
        grid=(NT_MOE,),
        in_specs=[
            pl.BlockSpec((BLK, D), lambda i, te, va: (va[i], 0)),
            pl.BlockSpec((1, D, F), lambda i, te, va: (te[va[i]], 0, 0)),
            pl.BlockSpec((1, 1, F), lambda i, te, va: (te[va[i]], 0, 0)),
            pl.BlockSpec((1, F, D), lambda i, te, va: (te[va[i]], 0, 0)),
            pl.BlockSpec((1, 1, D), lambda i, te, va: (te[va[i]], 0, 0)),
        ],
        out_specs=pl.BlockSpec((BLK, D), lambda i, te, va: (va[i], 0)),
    )
    return pl.pallas_call(
        _moe_gemm_body,
        grid_spec=grid_spec,
        out_shape=jax.ShapeDtypeStruct((PMOE, D), jnp.float32),
        compiler_params=pltpu.CompilerParams(vmem_limit_bytes=56 * 2**20),
    )(te, va, xs, w1, b1.reshape(E, 1, F), w2, b2.reshape(E, 1, D))


def _dense_body(x_ref, wd1_ref, bd1_ref, wd2_ref, bd2_ref, mf_ref, g0_ref,
                g1_ref, out_ref):
    f32 = jnp.float32
    h = jnp.maximum(
        jnp.dot(x_ref[...], wd1_ref[...], preferred_element_type=f32)
        + bd1_ref[...], 0.0)
    y = jnp.dot(h, wd2_ref[...], preferred_element_type=f32) + bd2_ref[...]
    out_ref[...] = (y * mf_ref[:, 2:3] + g0_ref[...] * mf_ref[:, 0:1]
                    + g1_ref[...] * mf_ref[:, 1:2])


def _dense_ffn(xf, wd1, bd1, wd2, bd2, mf, g0, g1):
    return pl.pallas_call(
        _dense_body,
        grid=(T // BLK,),
        in_specs=[
            pl.BlockSpec((BLK, D), lambda i: (i, 0)),
            pl.BlockSpec((D, F), lambda i: (0, 0)),
            pl.BlockSpec((1, F), lambda i: (0, 0)),
            pl.BlockSpec((F, D), lambda i: (0, 0)),
            pl.BlockSpec((1, D), lambda i: (0, 0)),
            pl.BlockSpec((BLK, E), lambda i: (i, 0)),
            pl.BlockSpec((BLK, D), lambda i: (i, 0)),
            pl.BlockSpec((BLK, D), lambda i: (i, 0)),
        ],
        out_specs=pl.BlockSpec((BLK, D), lambda i: (i, 0)),
        out_shape=jax.ShapeDtypeStruct((T, D), jnp.float32),
        compiler_params=pltpu.CompilerParams(vmem_limit_bytes=56 * 2**20),
    )(xf, wd1, bd1.reshape(1, F), wd2, bd2.reshape(1, D), mf, g0, g1)


def _pairgather_body(ys_hbm, mi_hbm, g0_hbm, g1_hbm, p0b, p1b, mchunk, buf,
                     sem, sem2):
    i32 = jnp.int32
    c = lax.axis_index("c")
    s = lax.axis_index("s")
    wid = c * 16 + s
    base = wid * TOK_W
    pltpu.sync_copy(mi_hbm.at[pl.ds(base * E, TOK_W * E)], mchunk)
    iota16 = lax.iota(i32, 16)
    for k in range(TOK_W // 16):
        i16 = (iota16 + k * 16) * E
        p0b[pl.ds(k * 16, 16)] = plsc.load_gather(mchunk, [i16])
        p1b[pl.ds(k * 16, 16)] = plsc.load_gather(mchunk, [i16 + 1])
    pltpu.async_copy(ys_hbm.at[p0b], buf, sem).wait()
    pltpu.sync_copy(buf, g0_hbm.at[pl.ds(base, TOK_W)])
    pltpu.async_copy(ys_hbm.at[p1b], buf, sem2).wait()
    pltpu.sync_copy(buf, g1_hbm.at[pl.ds(base, TOK_W)])


def _sc_pairgather(ys, mi_flat):
    f32 = jnp.float32
    i32 = jnp.int32
    mesh = plsc.VectorSubcoreMesh(core_axis_name="c", subcore_axis_name="s")
    kfn = pl.kernel(
        _pairgather_body,
        out_type=(jax.ShapeDtypeStruct((T, D), f32),
                  jax.ShapeDtypeStruct((T, D), f32)),
        mesh=mesh,
        scratch_types=[
            pltpu.VMEM((TOK_W,), i32),
            pltpu.VMEM((TOK_W,), i32),
            pltpu.VMEM((TOK_W * E,), i32),
            pltpu.VMEM((TOK_W, D), f32),
            pltpu.SemaphoreType.DMA,
            pltpu.SemaphoreType.DMA,
        ],
        compiler_params=pltpu.CompilerParams(needs_layout_passes=False),
    )
    return kfn(ys, mi_flat)


def kernel(x, Wg, bg, W1, b1, W2, b2, Wd1, bd1, Wd2, bd2, Wa1, ba1, Wa2, ba2):
    xf = x.reshape(T, D)
    mi, mf = _routing(xf, Wg, bg, Wa1, ba1, Wa2, ba2)
    te = mi[:NT_MOE, 2]
    va = mi[:NT_MOE, 3]
    mif = mi.reshape(T * E)
    xs = _sc_dispatch(xf, mif)
    ys = _moe_gemm(te, va, xs, W1, b1, W2, b2)
    g0, g1 = _sc_pairgather(ys, mif)
    out = _dense_ffn(xf, Wd1, bd1, Wd2, bd2, mf, g0, g1)
    return out.reshape(1, T, D)


# speedup vs baseline: 1.8723x; 1.0077x over previous
import functools

import jax
import jax.numpy as jnp
from jax import lax
from jax.experimental import pallas as pl
from jax.experimental.pallas import tpu as pltpu
from jax.experimental.pallas import tpu_sc as plsc

T = 2048
D = 1024
E = 8
F = 2048
BLK = 512
NT_MOE = 16
PMOE = NT_MOE * BLK
NW = 32
ROWS_W = PMOE // NW
TOK_W = T // NW


def _routing_body(x_ref, wg_ref, bg_ref, wa1_ref, ba1_ref, wa2_ref, ba2_ref,
                  mi_ref, mf_ref, oh1_ref, oh2_ref, r1_ref, r2_ref):
    f32 = jnp.float32
    xv = x_ref[...]
    logits = jnp.dot(xv, wg_ref[...], preferred_element_type=f32) + bg_ref[...]
    m = jnp.max(logits, axis=1, keepdims=True)
    ex = jnp.exp(logits - m)
    probs = ex / jnp.sum(ex, axis=1, keepdims=True)

    idx8 = lax.broadcasted_iota(jnp.int32, (T, E), 1).astype(f32)
    m1 = jnp.max(probs, axis=1, keepdims=True)
    i1 = jnp.min(jnp.where(probs >= m1, idx8, float(E)), axis=1, keepdims=True)
    oh1 = (idx8 == i1).astype(f32)
    pr2 = jnp.where(idx8 == i1, -1.0, probs)
    m2 = jnp.max(pr2, axis=1, keepdims=True)
    i2 = jnp.min(jnp.where(pr2 >= m2, idx8, float(E)), axis=1, keepdims=True)
    oh2 = (idx8 == i2).astype(f32)
    oh1_ref[...] = oh1
    oh2_ref[...] = oh2

    xavg = jnp.mean(xv, axis=0, keepdims=True)
    ha = jnp.maximum(
        jnp.dot(xavg, wa1_ref[...], preferred_element_type=f32) + ba1_ref[...], 0.0)
    z = jnp.dot(ha, wa2_ref[...], preferred_element_type=f32) + ba2_ref[...]
    r = 0.5 / (1.0 + jnp.exp(-z))
    sw = m1 + m2
    w1r = (m1 / sw) * r
    w2r = (m2 / sw) * r

    ls = (lax.broadcasted_iota(jnp.int32, (BLK, BLK), 0)
          > lax.broadcasted_iota(jnp.int32, (BLK, BLK), 1)).astype(f32)

    def body(cidx, carry):
        c1t, c2t = carry
        sl = pl.ds(cidx * BLK, BLK)
        b1 = oh1_ref[sl, :]
        b2 = oh2_ref[sl, :]
        cum1 = jnp.dot(ls, b1, preferred_element_type=f32) + c1t
        cum2 = jnp.dot(ls, b2, preferred_element_type=f32) + c2t
        r1_ref[sl, :] = jnp.sum(cum1 * b1, axis=1, keepdims=True)
        r2_ref[sl, :] = jnp.sum(cum2 * b2, axis=1, keepdims=True)
        return (c1t + jnp.sum(b1, axis=0, keepdims=True),
                c2t + jnp.sum(b2, axis=0, keepdims=True))

    c1t, c2t = lax.fori_loop(
        0, T // BLK, body,
        (jnp.zeros((1, E), f32), jnp.zeros((1, E), f32)))

    counts = c1t + c2t
    cap = jnp.floor((counts + float(BLK - 1)) / float(BLK)) * float(BLK)
    up8 = (lax.broadcasted_iota(jnp.int32, (E, E), 0)
           < lax.broadcasted_iota(jnp.int32, (E, E), 1)).astype(f32)
    gs = jnp.dot(cap, up8, preferred_element_type=f32)
    rank2 = r2_ref[...] + jnp.sum(oh2 * c1t, axis=1, keepdims=True)
    pos0 = jnp.sum(oh1 * gs, axis=1, keepdims=True) + r1_ref[...]
    pos1 = jnp.sum(oh2 * gs, axis=1, keepdims=True) + rank2
    jrowi = lax.broadcasted_iota(jnp.int32, (T, 1), 0).astype(f32)
    jrow = jrowi * float(BLK)
    te = jnp.sum((gs <= jrow).astype(f32), axis=1, keepdims=True) - 1.0
    tot = jnp.sum(cap, axis=1, keepdims=True)
    nact = jnp.floor((tot + float(BLK - 1)) / float(BLK))
    va = jnp.minimum(jrowi, nact - 1.0)

    mi_ref[...] = jnp.zeros((T, E), jnp.int32)
    mf_ref[...] = jnp.zeros((T, E), f32)
    mi_ref[:, 0:1] = pos0.astype(jnp.int32)
    mi_ref[:, 1:2] = pos1.astype(jnp.int32)
    mi_ref[:, 2:3] = te.astype(jnp.int32)
    mi_ref[:, 3:4] = va.astype(jnp.int32)
    mf_ref[:, 0:1] = w1r
    mf_ref[:, 1:2] = w2r
    mf_ref[:, 2:3] = (1.0 - r) + jnp.zeros((T, 1), f32)


def _routing(xf, wg, bg, wa1, ba1, wa2, ba2):
    f32 = jnp.float32
    return pl.pallas_call(
        _routing_body,
        out_shape=[jax.ShapeDtypeStruct((T, E), jnp.int32),
                   jax.ShapeDtypeStruct((T, E), f32)],
        scratch_shapes=[pltpu.VMEM((T, E), f32), pltpu.VMEM((T, E), f32),
                        pltpu.VMEM((T, 1), f32), pltpu.VMEM((T, 1), f32)],
    )(xf, wg, bg.reshape(1, E), wa1, ba1.reshape(1, 64), wa2, ba2.reshape(1, 1))


def _scdispatch_body(x_hbm, mi_hbm, xs_hbm, p0b, p1b, mchunk, xbuf, sem, sem2):
    i32 = jnp.int32
    c = lax.axis_index("c")
    s = lax.axis_index("s")
    wid = c * 16 + s
    base = wid * TOK_W
    pltpu.sync_copy(x_hbm.at[pl.ds(base, TOK_W)], xbuf)
    pltpu.sync_copy(mi_hbm.at[pl.ds(base * E, TOK_W * E)], mchunk)
    iota16 = lax.iota(i32, 16)
    for k in range(TOK_W // 16):
        i16 = (iota16 + k * 16) * E
        p0b[pl.ds(k * 16, 16)] = plsc.load_gather(mchunk, [i16])
        p1b[pl.ds(k * 16, 16)] = plsc.load_gather(mchunk, [i16 + 1])
    a = pltpu.async_copy(xbuf, xs_hbm.at[p0b], sem)
    b = pltpu.async_copy(xbuf, xs_hbm.at[p1b], sem2)
    a.wait()
    b.wait()


def _sc_dispatch(xf, mi_flat):
    f32 = jnp.float32
    i32 = jnp.int32
    mesh = plsc.VectorSubcoreMesh(core_axis_name="c", subcore_axis_name="s")
    kfn = pl.kernel(
        _scdispatch_body,
        out_type=jax.ShapeDtypeStruct((PMOE, D), f32),
        mesh=mesh,
        scratch_types=[
            pltpu.VMEM((TOK_W,), i32),
            pltpu.VMEM((TOK_W,), i32),
            pltpu.VMEM((TOK_W * E,), i32),
            pltpu.VMEM((TOK_W, D), f32),
            pltpu.SemaphoreType.DMA,
            pltpu.SemaphoreType.DMA,
        ],
        compiler_params=pltpu.CompilerParams(needs_layout_passes=False),
    )
    return kfn(xf, mi_flat)


def _moe_gemm_body(te_ref, va_ref, xs_ref, w1_ref, b1_ref, w2_ref, b2_ref,
                   out_ref):
    f32 = jnp.float32
    i = pl.program_id(0)

    @pl.when(va_ref[i] == i)
    def _compute():
        h = jnp.maximum(
            jnp.dot(xs_ref[...], w1_ref[0], preferred_element_type=f32)
            + b1_ref[0], 0.0)
        out_ref[...] = (jnp.dot(h, w2_ref[0], preferred_element_type=f32)
                        + b2_ref[0])


def _moe_gemm(te, va, xs, w1, b1, w2, b2):
    grid_spec = pltpu.PrefetchScalarGridSpec(
        num_scalar_prefetch=2,
        grid=(NT_MOE,),
        in_specs=[
            pl.BlockSpec((BLK, D), lambda i, te, va: (va[i], 0)),
            pl.BlockSpec((1, D, F), lambda i, te, va: (te[va[i]], 0, 0)),
            pl.BlockSpec((1, 1, F), lambda i, te, va: (te[va[i]], 0, 0)),
            pl.BlockSpec((1, F, D), lambda i, te, va: (te[va[i]], 0, 0)),
            pl.BlockSpec((1, 1, D), lambda i, te, va: (te[va[i]], 0, 0)),
        ],
        out_specs=pl.BlockSpec((BLK, D), lambda i, te, va: (va[i], 0)),
    )
    return pl.pallas_call(
        _moe_gemm_body,
        grid_spec=grid_spec,
        out_shape=jax.ShapeDtypeStruct((PMOE, D), jnp.float32),
        compiler_params=pltpu.CompilerParams(vmem_limit_bytes=56 * 2**20),
    )(te, va, xs, w1, b1.reshape(E, 1, F), w2, b2.reshape(E, 1, D))


def _dense_body(x_ref, wd1_ref, bd1_ref, wd2_ref, bd2_ref, out_ref):
    f32 = jnp.float32
    h = jnp.maximum(
        jnp.dot(x_ref[...], wd1_ref[...], preferred_element_type=f32)
        + bd1_ref[...], 0.0)
    out_ref[...] = (jnp.dot(h, wd2_ref[...], preferred_element_type=f32)
                    + bd2_ref[...])


def _dense_ffn(xf, wd1, bd1, wd2, bd2):
    return pl.pallas_call(
        _dense_body,
        grid=(T // BLK,),
        in_specs=[
            pl.BlockSpec((BLK, D), lambda i: (i, 0)),
            pl.BlockSpec((D, F), lambda i: (0, 0)),
            pl.BlockSpec((1, F), lambda i: (0, 0)),
            pl.BlockSpec((F, D), lambda i: (0, 0)),
            pl.BlockSpec((1, D), lambda i: (0, 0)),
        ],
        out_specs=pl.BlockSpec((BLK, D), lambda i: (i, 0)),
        out_shape=jax.ShapeDtypeStruct((T, D), jnp.float32),
        compiler_params=pltpu.CompilerParams(vmem_limit_bytes=56 * 2**20),
    )(xf, wd1, bd1.reshape(1, F), wd2, bd2.reshape(1, D))


def _combine_body(yd_ref, g0_ref, g1_ref, mf_ref, out_ref):
    out_ref[...] = (yd_ref[...] * mf_ref[:, 2:3] + g0_ref[...] * mf_ref[:, 0:1]
                    + g1_ref[...] * mf_ref[:, 1:2])


def _combine(yd, g0, g1, mf):
    return pl.pallas_call(
        _combine_body,
        grid=(T // BLK,),
        in_specs=[
            pl.BlockSpec((BLK, D), lambda i: (i, 0)),
            pl.BlockSpec((BLK, D), lambda i: (i, 0)),
            pl.BlockSpec((BLK, D), lambda i: (i, 0)),
            pl.BlockSpec((BLK, E), lambda i: (i, 0)),
        ],
        out_specs=pl.BlockSpec((BLK, D), lambda i: (i, 0)),
        out_shape=jax.ShapeDtypeStruct((T, D), jnp.float32),
    )(yd, g0, g1, mf)


def _pairgather_body(ys_hbm, mi_hbm, g0_hbm, g1_hbm, p0b, p1b, mchunk, buf,
                     sem, sem2):
    i32 = jnp.int32
    c = lax.axis_index("c")
    s = lax.axis_index("s")
    wid = c * 16 + s
    base = wid * TOK_W
    pltpu.sync_copy(mi_hbm.at[pl.ds(base * E, TOK_W * E)], mchunk)
    iota16 = lax.iota(i32, 16)
    for k in range(TOK_W // 16):
        i16 = (iota16 + k * 16) * E
        p0b[pl.ds(k * 16, 16)] = plsc.load_gather(mchunk, [i16])
        p1b[pl.ds(k * 16, 16)] = plsc.load_gather(mchunk, [i16 + 1])
    pltpu.async_copy(ys_hbm.at[p0b], buf, sem).wait()
    pltpu.sync_copy(buf, g0_hbm.at[pl.ds(base, TOK_W)])
    pltpu.async_copy(ys_hbm.at[p1b], buf, sem2).wait()
    pltpu.sync_copy(buf, g1_hbm.at[pl.ds(base, TOK_W)])


def _sc_pairgather(ys, mi_flat):
    f32 = jnp.float32
    i32 = jnp.int32
    mesh = plsc.VectorSubcoreMesh(core_axis_name="c", subcore_axis_name="s")
    kfn = pl.kernel(
        _pairgather_body,
        out_type=(jax.ShapeDtypeStruct((T, D), f32),
                  jax.ShapeDtypeStruct((T, D), f32)),
        mesh=mesh,
        scratch_types=[
            pltpu.VMEM((TOK_W,), i32),
            pltpu.VMEM((TOK_W,), i32),
            pltpu.VMEM((TOK_W * E,), i32),
            pltpu.VMEM((TOK_W, D), f32),
            pltpu.SemaphoreType.DMA,
            pltpu.SemaphoreType.DMA,
        ],
        compiler_params=pltpu.CompilerParams(needs_layout_passes=False),
    )
    return kfn(ys, mi_flat)


def kernel(x, Wg, bg, W1, b1, W2, b2, Wd1, bd1, Wd2, bd2, Wa1, ba1, Wa2, ba2):
    xf = x.reshape(T, D)
    mi, mf = _routing(xf, Wg, bg, Wa1, ba1, Wa2, ba2)
    te = mi[:NT_MOE, 2]
    va = mi[:NT_MOE, 3]
    mif = mi.reshape(T * E)
    xs = _sc_dispatch(xf, mif)
    yd = _dense_ffn(xf, Wd1, bd1, Wd2, bd2)
    ys = _moe_gemm(te, va, xs, W1, b1, W2, b2)
    g0, g1 = _sc_pairgather(ys, mif)
    out = _combine(yd, g0, g1, mf)
    return out.reshape(1, T, D)
